# Initial kernel scaffold; baseline (speedup 1.0000x reference)
import jax
import jax.numpy as jnp
from jax.experimental import pallas as pl


def kernel(x, edge_index, Wl0, bl0, Wr0, w_gate, eWl, ebl, eWr, Wl2, bl2, Wr2, Wl3, bl3, Wr3):
    raise NotImplementedError("write your pallas kernel here")


# speedup vs baseline: 6.1978x; 6.1978x over previous
import functools

import jax
import jax.numpy as jnp
from jax import lax
from jax.experimental import pallas as pl
from jax.experimental.pallas import tpu as pltpu
from jax.experimental.pallas import tpu_sc as plsc

N = 10000
E_EDGES = 320000
D = 128
NUM_EXPERTS = 8

NC = 2
NS = 16
NW = NC * NS
EPW = E_EDGES // NW
CHUNK = 80
NCHUNK = EPW // CHUNK
NP = 10240
ROWS_PER_TILE = NP // NS
DEGW = 128

_f32 = jnp.float32


def _make_agg():
  mesh = plsc.VectorSubcoreMesh(core_axis_name="c", subcore_axis_name="s")

  @functools.partial(
      pl.kernel,
      mesh=mesh,
      out_type=jax.ShapeDtypeStruct((NC, NP, D), _f32),
      scratch_types=[
          pltpu.VMEM((NCHUNK, CHUNK), jnp.int32),
          pltpu.VMEM((NCHUNK, CHUNK), jnp.int32),
          pltpu.VMEM((CHUNK, D), _f32),
          pltpu.VMEM_SHARED((NP, D), _f32),
          pltpu.SemaphoreType.DMA,
      ],
  )
  def agg(x_hbm, src_hbm, dst_hbm, zero_hbm, out_hbm,
          src_v, dst_v, rows_v, acc_sh, sem):
    cid = lax.axis_index("c")
    sid = lax.axis_index("s")
    wid = sid * NC + cid
    rsl = pl.ds(sid * ROWS_PER_TILE, ROWS_PER_TILE)

    pltpu.sync_copy(src_hbm.at[wid], src_v)
    pltpu.sync_copy(dst_hbm.at[wid], dst_v)
    pltpu.sync_copy(zero_hbm.at[rsl], acc_sh.at[rsl])
    plsc.subcore_barrier()

    def body(j, carry):
      pltpu.async_copy(x_hbm.at[src_v.at[j]], rows_v, sem).wait()
      pltpu.sync_copy(rows_v, acc_sh.at[dst_v.at[j]], add=True)
      return carry

    lax.fori_loop(0, NCHUNK, body, 0)
    plsc.subcore_barrier()

    pltpu.sync_copy(acc_sh.at[rsl], out_hbm.at[cid, rsl])

  return agg


def _make_deg():
  mesh = plsc.VectorSubcoreMesh(core_axis_name="c", subcore_axis_name="s")

  @functools.partial(
      pl.kernel,
      mesh=mesh,
      out_type=jax.ShapeDtypeStruct((NC, NP, DEGW), _f32),
      scratch_types=[
          pltpu.VMEM((NCHUNK, CHUNK), jnp.int32),
          pltpu.VMEM((CHUNK, DEGW), _f32),
          pltpu.VMEM_SHARED((NP, DEGW), _f32),
      ],
  )
  def deg(dst_hbm, zdeg_hbm, ones_hbm, out_hbm, dst_v, ones_v, deg_sh):
    cid = lax.axis_index("c")
    sid = lax.axis_index("s")
    wid = sid * NC + cid
    rsl = pl.ds(sid * ROWS_PER_TILE, ROWS_PER_TILE)

    pltpu.sync_copy(dst_hbm.at[wid], dst_v)
    pltpu.sync_copy(ones_hbm, ones_v)
    pltpu.sync_copy(zdeg_hbm.at[rsl], deg_sh.at[rsl])
    plsc.subcore_barrier()

    def body(j, carry):
      pltpu.sync_copy(ones_v, deg_sh.at[dst_v.at[j]], add=True)
      return carry

    lax.fori_loop(0, NCHUNK, body, 0)
    plsc.subcore_barrier()
    pltpu.sync_copy(deg_sh.at[rsl], out_hbm.at[cid, rsl])

  return deg


@functools.cache
def _get_agg():
  return _make_agg()


@functools.cache
def _get_deg():
  return _make_deg()


def _agg(*args):
  return _get_agg()(*args)


def _deg(*args):
  return _get_deg()(*args)


def _inv_deg(degp_ref):
  deg = degp_ref[0, :, 0:1] + degp_ref[1, :, 0:1]
  return 1.0 / jnp.maximum(deg, 1.0)


def _dot_t(a, w):
  return lax.dot_general(a, w, (((1,), (1,)), ((), ())),
                         preferred_element_type=_f32)


def _layer_body(apply_relu, p_ref, degp_ref, h_ref, wl_ref, bl_ref, wr_ref,
                o_ref):
  a = (p_ref[0] + p_ref[1]) * _inv_deg(degp_ref)
  y = _dot_t(a, wl_ref[...]) + bl_ref[...] + _dot_t(h_ref[...], wr_ref[...])
  o_ref[...] = jnp.maximum(y, 0.0) if apply_relu else y


def _moe_body(p_ref, degp_ref, h_ref, wg_ref, ewl_ref, ebl_ref, ewr_ref,
              o_ref):
  a = (p_ref[0] + p_ref[1]) * _inv_deg(degp_ref)
  h = h_ref[...]
  logits = jnp.dot(h, wg_ref[...], preferred_element_type=_f32)
  iota = lax.broadcasted_iota(jnp.int32, logits.shape, 1)
  m1 = jnp.max(logits, axis=1, keepdims=True)
  e1 = jnp.min(jnp.where(logits == m1, iota, NUM_EXPERTS), axis=1,
               keepdims=True)
  oh1 = iota == e1
  masked = jnp.where(oh1, -jnp.inf, logits)
  m2 = jnp.max(masked, axis=1, keepdims=True)
  e2 = jnp.min(jnp.where(masked == m2, iota, NUM_EXPERTS), axis=1,
               keepdims=True)
  oh2 = iota == e2
  g1 = 1.0 / (1.0 + jnp.exp(m2 - m1))
  gates = g1 * oh1.astype(_f32) + (1.0 - g1) * oh2.astype(_f32)

  acc = jnp.dot(gates, ebl_ref[...], preferred_element_type=_f32)
  for e in range(NUM_EXPERTS):
    ge = gates[:, e:e + 1]
    acc = acc + ge * (_dot_t(a, ewl_ref[e]) + _dot_t(h, ewr_ref[e]))
  o_ref[...] = jnp.maximum(acc, 0.0)


_TB = 1000


def _row_spec(shape):
  if len(shape) == 3:
    return pl.BlockSpec((shape[0], _TB, shape[2]), lambda i: (0, i, 0))
  return pl.BlockSpec((_TB, shape[1]), lambda i: (i, 0))


def _full_spec(shape):
  nd = len(shape)
  return pl.BlockSpec(shape, lambda i: (0,) * nd)


def _tc_call(body, row_args, full_args):
  in_specs = ([_row_spec(a.shape) for a in row_args] +
              [_full_spec(a.shape) for a in full_args])
  return pl.pallas_call(
      body,
      grid=(N // _TB,),
      in_specs=in_specs,
      out_specs=pl.BlockSpec((_TB, D), lambda i: (i, 0)),
      out_shape=jax.ShapeDtypeStruct((N, D), _f32),
  )(*row_args, *full_args)


def _layer(p, degp, h, wl, bl, wr, apply_relu):
  return _tc_call(functools.partial(_layer_body, apply_relu),
                  [p, degp, h], [wl, bl, wr])


def _moe(p, degp, h, wg, ewl, ebl, ewr):
  return _tc_call(_moe_body, [p, degp, h], [wg, ewl, ebl, ewr])


def kernel(x, edge_index, Wl0, bl0, Wr0, w_gate, eWl, ebl, eWr,
           Wl2, bl2, Wr2, Wl3, bl3, Wr3):
  src = edge_index[0].astype(jnp.int32).reshape(NW, NCHUNK, CHUNK)
  dst = edge_index[1].astype(jnp.int32).reshape(NW, NCHUNK, CHUNK)
  zero = jnp.zeros((NP, D), _f32)
  zdeg = jnp.zeros((NP, DEGW), _f32)
  ones = jnp.ones((CHUNK, DEGW), _f32)
  bl0r = bl0.reshape(1, D)
  bl2r = bl2.reshape(1, D)
  bl3r = bl3.reshape(1, D)

  degp = _deg(dst, zdeg, ones)
  p0 = _agg(x, src, dst, zero)
  h1 = _layer(p0, degp, x, Wl0, bl0r, Wr0, True)
  p1 = _agg(h1, src, dst, zero)
  h2 = _moe(p1, degp, h1, w_gate, eWl, ebl, eWr)
  p2 = _agg(h2, src, dst, zero)
  h3 = _layer(p2, degp, h2, Wl2, bl2r, Wr2, True)
  p3 = _agg(h3, src, dst, zero)
  return _layer(p3, degp, h3, Wl3, bl3r, Wr3, False)
